# Initial kernel scaffold; baseline (speedup 1.0000x reference)
import jax
import jax.numpy as jnp
from jax.experimental import pallas as pl


def kernel(x, edge_index, batch, W_gcn, b_gcn, bn1_g, bn1_b, bn2_g, bn2_b, bn3_g, bn3_b, ln1_g, ln1_b, ln2_g, ln2_b, W11, b11, W12, b12):
    raise NotImplementedError("write your pallas kernel here")


# speedup vs baseline: 143.6041x; 143.6041x over previous
import functools

import jax
import jax.numpy as jnp
from jax import lax
from jax.experimental import pallas as pl
from jax.experimental.pallas import tpu as pltpu
from jax.experimental.pallas import tpu_sc as plsc

N = 10000
E = 320000
C = 128
B = 200
BP = 256
EPS_BN = 1e-5
EPS_LN = 1e-5

NC = 2
NS = 16
NW = NC * NS
L = 16
EPW = E // NW
NPW = 320
GPW = 7

_mesh = plsc.VectorSubcoreMesh(core_axis_name="c", subcore_axis_name="s")


def _wid():
    return lax.axis_index("s") * NC + lax.axis_index("c")


def _sload(ref, i):
    idx = jnp.full((L,), i, jnp.int32)
    return plsc.load_gather(ref, [idx])[0]


def _zero_ref(ref, nvec, value=0.0):
    fill = jnp.full((L,), value, jnp.float32)

    def body(i, _):
        ref[pl.ds(i * L, L)] = fill
        return 0

    lax.fori_loop(0, nvec, body, 0)


def _s1_body(dst_hbm, batch_hbm, degp_hbm, bhist_hbm, dst_v, batch_v, deg_v, bh_v):
    w = _wid()
    _zero_ref(deg_v, N // L)
    _zero_ref(bh_v, BP // L)

    pltpu.sync_copy(dst_hbm.at[pl.ds(w * EPW, EPW)], dst_v)
    pltpu.sync_copy(batch_hbm, batch_v.at[pl.ds(0, N)])

    ones = jnp.ones((L,), jnp.float32)

    def deg_loop(i, _):
        idx = dst_v[pl.ds(i * L, L)]
        plsc.addupdate_scatter(deg_v, [idx], ones)
        return 0

    lax.fori_loop(0, EPW // L, deg_loop, 0)

    iota = lax.iota(jnp.int32, L)

    def bh_loop(i, _):
        base = w * NPW + i * L
        idx = batch_v[pl.ds(base, L)]
        m = (base + iota) < N
        idx = jnp.where(m, idx, 0)
        plsc.addupdate_scatter(bh_v, [idx], ones, mask=m)
        return 0

    lax.fori_loop(0, NPW // L, bh_loop, 0)

    pltpu.sync_copy(deg_v, degp_hbm.at[w])
    pltpu.sync_copy(bh_v, bhist_hbm.at[w])


_s1 = pl.kernel(
    _s1_body,
    out_type=(
        jax.ShapeDtypeStruct((NW, N), jnp.float32),
        jax.ShapeDtypeStruct((NW, BP), jnp.float32),
    ),
    mesh=_mesh,
    compiler_params=pltpu.CompilerParams(needs_layout_passes=False),
    scratch_types=[
        pltpu.VMEM((EPW,), jnp.int32),
        pltpu.VMEM((NW * NPW,), jnp.int32),
        pltpu.VMEM((N,), jnp.float32),
        pltpu.VMEM((BP,), jnp.float32),
    ],
)


def _k2_body(x_ref, w_ref, degp_ref, bh_ref,
             dis_ref, g_ref, selfc_ref, starts_ref, counts_ref, kk_ref):
    xb = x_ref[...].astype(jnp.bfloat16).astype(jnp.float32)
    wv = w_ref[...][:, 0].astype(jnp.bfloat16).astype(jnp.float32)
    h = jnp.sum(xb * wv[None, :], axis=1)
    deg = jnp.sum(degp_ref[...], axis=0) + 1.0
    r0 = lax.rsqrt(deg)
    dis = r0 * (1.5 - 0.5 * deg * r0 * r0)
    dis_ref[...] = dis
    g_ref[...] = dis * h
    selfc_ref[...] = dis * dis * h

    cnt = jnp.sum(bh_ref[...], axis=0)
    counts_ref[...] = cnt.astype(jnp.int32)
    kk_ref[...] = jnp.ceil(jnp.float32(0.3) * cnt).astype(jnp.int32)
    row = lax.broadcasted_iota(jnp.int32, (BP, BP), 0)
    col = lax.broadcasted_iota(jnp.int32, (BP, BP), 1)
    tri = (col < row).astype(jnp.float32)
    starts = jnp.dot(tri, cnt[:, None], preferred_element_type=jnp.float32)
    starts_ref[...] = starts[:, 0].astype(jnp.int32)


_k2 = pl.pallas_call(
    _k2_body,
    out_shape=(
        jax.ShapeDtypeStruct((N,), jnp.float32),
        jax.ShapeDtypeStruct((N,), jnp.float32),
        jax.ShapeDtypeStruct((N,), jnp.float32),
        jax.ShapeDtypeStruct((BP,), jnp.int32),
        jax.ShapeDtypeStruct((BP,), jnp.int32),
        jax.ShapeDtypeStruct((BP,), jnp.int32),
    ),
)


def _s2_body(src_hbm, dst_hbm, g_hbm, dis_hbm, scp_hbm,
             src_v, dst_v, g_v, dis_v, acc_v):
    w = _wid()
    _zero_ref(acc_v, N // L)
    pltpu.sync_copy(src_hbm.at[pl.ds(w * EPW, EPW)], src_v)
    pltpu.sync_copy(dst_hbm.at[pl.ds(w * EPW, EPW)], dst_v)
    pltpu.sync_copy(g_hbm, g_v)
    pltpu.sync_copy(dis_hbm, dis_v)

    def loop(i, _):
        s = src_v[pl.ds(i * L, L)]
        d = dst_v[pl.ds(i * L, L)]
        val = plsc.load_gather(g_v, [s]) * plsc.load_gather(dis_v, [d])
        plsc.addupdate_scatter(acc_v, [d], val)
        return 0

    lax.fori_loop(0, EPW // L, loop, 0)
    pltpu.sync_copy(acc_v, scp_hbm.at[w])


_s2 = pl.kernel(
    _s2_body,
    out_type=jax.ShapeDtypeStruct((NW, N), jnp.float32),
    mesh=_mesh,
    compiler_params=pltpu.CompilerParams(needs_layout_passes=False),
    scratch_types=[
        pltpu.VMEM((EPW,), jnp.int32),
        pltpu.VMEM((EPW,), jnp.int32),
        pltpu.VMEM((N,), jnp.float32),
        pltpu.VMEM((N,), jnp.float32),
        pltpu.VMEM((N,), jnp.float32),
    ],
)


def _k3_body(scp_ref, selfc_ref, b_ref, score_ref):
    score_ref[...] = jnp.sum(scp_ref[...], axis=0) + selfc_ref[...] + b_ref[0, 0]


_k3 = pl.pallas_call(
    _k3_body,
    out_shape=jax.ShapeDtypeStruct((N,), jnp.float32),
)


def _s3_body(score_hbm, batch_hbm, starts_hbm, counts_hbm, kk_hbm, x_hbm,
             psum_hbm, pmax_hbm,
             score_v, batch_v, sel_v, meta_v, xc_v, accs_v, accm_v):
    w = _wid()
    g0 = jnp.minimum(w * GPW, B)
    g1 = jnp.minimum(w * GPW + GPW, B)

    pltpu.sync_copy(score_hbm, score_v)
    pltpu.sync_copy(batch_hbm, batch_v)
    pltpu.sync_copy(starts_hbm, meta_v.at[pl.ds(0, BP)])
    pltpu.sync_copy(counts_hbm, meta_v.at[pl.ds(BP, BP)])
    pltpu.sync_copy(kk_hbm, meta_v.at[pl.ds(2 * BP, BP)])

    _zero_ref(sel_v, N // L)
    _zero_ref(accs_v, GPW * C // L)
    _zero_ref(accm_v, GPW * C // L, value=-jnp.inf)

    iota = lax.iota(jnp.int32, L)

    def graph_loop(g, _):
        a = _sload(meta_v, g)
        n = _sload(meta_v, BP + g)
        k = _sload(meta_v, 2 * BP + g)
        nv = (n + L - 1) // L

        def node_loop(i, _):
            si = _sload(score_v, a + i)
            si_v = jnp.full((L,), si)

            def vec_loop(v, carry):
                gt, eq = carry
                pos = v * L + iota
                valid = pos < n
                idx = jnp.where(valid, a + pos, 0)
                sv = plsc.load_gather(score_v, [idx])
                gt = gt + plsc.all_reduce_population_count((sv > si_v) & valid)
                eq = eq + plsc.all_reduce_population_count(
                    (sv == si_v) & valid & (pos < i))
                return gt, eq

            zero = jnp.zeros((L,), jnp.int32)
            gt, eq = lax.fori_loop(0, nv, vec_loop, (zero, zero))
            selected = (gt + eq) < k
            selv = jnp.where(selected, 1.0, 0.0)
            plsc.store_scatter(sel_v, [jnp.full((L,), a + i)], selv,
                               mask=(iota == 0))
            return 0

        lax.fori_loop(0, n, node_loop, 0)
        return 0

    lax.fori_loop(g0, g1, graph_loop, 0)

    a0 = _sload(meta_v, g0)
    a1 = _sload(meta_v, g1)
    t0 = a0 // L
    t1 = (a1 + L - 1) // L

    def chunk_loop(t, _):
        pltpu.sync_copy(x_hbm.at[pl.ds(t * L, L)], xc_v)
        sel16 = sel_v[pl.ds(t * L, L)]
        b16 = batch_v[pl.ds(t * L, L)]
        for r in range(L):
            sel = sel16[r]

            @pl.when(sel > 0.5)
            def _do():
                gg = b16[r]
                base = (gg - g0) * C
                for v in range(C // L):
                    xv = xc_v[r, pl.ds(v * L, L)]
                    sl = pl.ds(base + v * L, L)
                    accs_v[sl] = accs_v[sl] + xv
                    accm_v[sl] = jnp.maximum(accm_v[sl], xv)
        return 0

    lax.fori_loop(t0, t1, chunk_loop, 0)

    def out_loop(j, _):
        pltpu.sync_copy(accs_v.at[pl.ds(j * C, C)], psum_hbm.at[g0 + j])
        pltpu.sync_copy(accm_v.at[pl.ds(j * C, C)], pmax_hbm.at[g0 + j])
        return 0

    lax.fori_loop(0, g1 - g0, out_loop, 0)


_s3 = pl.kernel(
    _s3_body,
    out_type=(
        jax.ShapeDtypeStruct((B, C), jnp.float32),
        jax.ShapeDtypeStruct((B, C), jnp.float32),
    ),
    mesh=_mesh,
    compiler_params=pltpu.CompilerParams(needs_layout_passes=False),
    scratch_types=[
        pltpu.VMEM((N,), jnp.float32),
        pltpu.VMEM((N,), jnp.int32),
        pltpu.VMEM((N,), jnp.float32),
        pltpu.VMEM((3 * BP,), jnp.int32),
        pltpu.VMEM((L, C), jnp.float32),
        pltpu.VMEM((GPW * C,), jnp.float32),
        pltpu.VMEM((GPW * C,), jnp.float32),
    ],
)


def _k4_body(psum_ref, pmax_ref, kk_ref,
             bn1g_ref, bn1b_ref, bn2g_ref, bn2b_ref, bn3g_ref, bn3b_ref,
             ln1g_ref, ln1b_ref, ln2g_ref, ln2b_ref,
             w11_ref, b11_ref, w12_ref, b12_ref,
             o0_ref, o1_ref, o2_ref):
    s = psum_ref[...]
    kf = kk_ref[...].astype(jnp.float32)[:, None]
    mean = s / jnp.maximum(kf, 1.0)
    mx = jnp.where(kf > 0.0, pmax_ref[...], 0.0)

    bscale = 1.0 / jnp.sqrt(jnp.float32(1.0 + EPS_BN))

    def bn(t, g, b):
        return t * bscale * g[None, :] + b[None, :]

    ln1g = ln1g_ref[...]
    ln1b = ln1b_ref[...]
    ln2g = ln2g_ref[...]
    ln2b = ln2b_ref[...]
    w11 = w11_ref[...]
    b11 = b11_ref[...]
    w12 = w12_ref[...]
    b12 = b12_ref[...]

    def ln(t, g, b):
        m = jnp.mean(t, axis=1, keepdims=True)
        v = jnp.mean((t - m) ** 2, axis=1, keepdims=True)
        return (t - m) / jnp.sqrt(v + EPS_LN) * g[None, :] + b[None, :]

    def branch(xs, o_ref):
        h = ln(xs, ln1g, ln1b)
        h = jnp.maximum(jnp.dot(h, w11, preferred_element_type=jnp.float32)
                        + b11[None, :], 0.0)
        h = jnp.dot(h, w12, preferred_element_type=jnp.float32) + b12[None, :]
        o_ref[...] = ln(h + xs, ln2g, ln2b)

    branch(bn(mx, bn1g_ref[...], bn1b_ref[...]), o0_ref)
    branch(bn(mean, bn2g_ref[...], bn2b_ref[...]), o1_ref)
    branch(bn(s, bn3g_ref[...], bn3b_ref[...]), o2_ref)


_k4 = pl.pallas_call(
    _k4_body,
    out_shape=(
        jax.ShapeDtypeStruct((B, C), jnp.float32),
        jax.ShapeDtypeStruct((B, C), jnp.float32),
        jax.ShapeDtypeStruct((B, C), jnp.float32),
    ),
)


def kernel(x, edge_index, batch, W_gcn, b_gcn, bn1_g, bn1_b, bn2_g, bn2_b,
           bn3_g, bn3_b, ln1_g, ln1_b, ln2_g, ln2_b, W11, b11, W12, b12):
    src = edge_index[0]
    dst = edge_index[1]

    degp, bhist = _s1(dst, batch)
    dis, gvec, selfc, starts, counts, kk = _k2(x, W_gcn, degp, bhist)
    scp = _s2(src, dst, gvec, dis)
    score = _k3(scp, selfc, b_gcn.reshape(1, 1))
    psum, pmax = _s3(score, batch, starts, counts, kk, x)
    o0, o1, o2 = _k4(psum, pmax, kk[:B],
                     bn1_g, bn1_b, bn2_g, bn2_b, bn3_g, bn3_b,
                     ln1_g, ln1_b, ln2_g, ln2_b, W11, b11, W12, b12)
    return jnp.stack([o0, o1, o2], axis=1)
